# baseline (speedup 1.0000x reference)
import functools

import jax
import jax.numpy as jnp
from jax import lax
from jax.experimental import pallas as pl
from jax.experimental.pallas import tpu as pltpu
from jax.experimental.pallas import tpu_sc as plsc

BATCH = 16384
HIDDEN = 64

_info = plsc.get_sparse_core_info()
_NC, _NS, _L = _info.num_cores, _info.num_subcores, _info.num_lanes
_NW = _NC * _NS
_B_PER_W = BATCH // _NW


def _bi_embed_kernel(kidx_hbm, vidx_hbm, key_hbm, val_hbm, out_hbm,
                     kidx_v, vidx_v, krows, vrows, sem_k, sem_v):
    wid = lax.axis_index("s") * _NC + lax.axis_index("c")
    base = wid * _B_PER_W

    pltpu.sync_copy(kidx_hbm.at[pl.ds(base, _B_PER_W)], kidx_v)
    pltpu.sync_copy(vidx_hbm.at[pl.ds(base, _B_PER_W)], vidx_v)

    cp_k = pltpu.async_copy(key_hbm.at[kidx_v], krows, sem_k)
    cp_v = pltpu.async_copy(val_hbm.at[vidx_v], vrows, sem_v)
    cp_k.wait()
    cp_v.wait()

    def add_row(i, carry):
        for d in range(HIDDEN // _L):
            sl = (i, pl.ds(d * _L, _L))
            krows[sl] = krows[sl] + vrows[sl]
        return carry

    lax.fori_loop(0, _B_PER_W, add_row, 0)

    pltpu.sync_copy(krows, out_hbm.at[pl.ds(base, _B_PER_W)])


@jax.jit
def kernel(x, key_table, val_table):
    x_key = x[:, 0]
    x_val = x[:, 1]
    run = pl.kernel(
        _bi_embed_kernel,
        mesh=plsc.VectorSubcoreMesh(core_axis_name="c", subcore_axis_name="s"),
        out_type=jax.ShapeDtypeStruct((BATCH, HIDDEN), jnp.float32),
        scratch_types=[
            pltpu.VMEM((_B_PER_W,), jnp.int32),
            pltpu.VMEM((_B_PER_W,), jnp.int32),
            pltpu.VMEM((_B_PER_W, HIDDEN), jnp.float32),
            pltpu.VMEM((_B_PER_W, HIDDEN), jnp.float32),
            pltpu.SemaphoreType.DMA,
            pltpu.SemaphoreType.DMA,
        ],
        compiler_params=pltpu.CompilerParams(use_tc_tiling_on_sc=False),
    )
    return run(x_key, x_val, key_table, val_table)


# speedup vs baseline: 1.9195x; 1.9195x over previous
import jax
import jax.numpy as jnp
from jax import lax
from jax.experimental import pallas as pl
from jax.experimental.pallas import tpu as pltpu
from jax.experimental.pallas import tpu_sc as plsc

BATCH = 16384
HIDDEN = 64
DIM = 1000001

_info = plsc.get_sparse_core_info()
_NC, _NS, _L = _info.num_cores, _info.num_subcores, _info.num_lanes

NPAGES = (DIM + 127) // 128
PT = (NPAGES + _NS - 1) // _NS
NCHUNK = BATCH // _L
BUCKET = 24224
ACCR = BATCH // 2
ACCR_PAD = ACCR + _L
CNT = 512


def _sc_body(kidx_hbm, vidx_hbm, kt_hbm, vt_hbm, part_hbm,
             idx_v, bucket, cnts, starts, offs, rows, sb3, zbuf,
             acc, sem_i, sem_a, sem_b, sem_o):
    c = lax.axis_index("c")
    s = lax.axis_index("s")
    iota = lax.iota(jnp.int32, _L)
    lo = s * PT

    for h in range(_L):
        for j in range(128 // _L):
            zbuf[h, pl.ds(j * _L, _L)] = jnp.zeros((_L,), jnp.float32)
    rows_per_tile = ACCR // _NS
    for k in range(rows_per_tile // _L):
        dst = pl.multiple_of(s * rows_per_tile + k * _L, _L)
        pltpu.sync_copy(zbuf, acc.at[pl.ds(dst, _L), :])

    @pl.when(s == 0)
    def _():
        pltpu.sync_copy(zbuf, acc.at[pl.ds(ACCR, _L), :])

    @pl.when(c == 0)
    def _():
        pltpu.async_copy(kidx_hbm, idx_v, sem_i)

    @pl.when(c != 0)
    def _():
        pltpu.async_copy(vidx_hbm, idx_v, sem_i)

    pltpu.make_async_copy(kidx_hbm, idx_v, sem_i).wait()

    for j in range(CNT // _L):
        cnts[pl.ds(j * _L, _L)] = jnp.zeros((_L,), jnp.int32)

    def hist_body(k, carry):
        v = idx_v[pl.ds(k * _L, _L)]
        lp = lax.shift_right_logical(v, 7) - lo
        m = (lp >= 0) & (lp < PT)
        dup, lastm = plsc.scan_count(lp, mask=m)
        plsc.addupdate_scatter(cnts, [lp], dup, mask=m & lastm)
        return carry

    lax.fori_loop(0, NCHUNK, hist_body, jnp.int32(0))

    def cum_body(i, carry):
        v = cnts[pl.ds(i * _L, _L)]
        vp = lax.bitwise_and(v + (_L - 1), ~(_L - 1))
        ic = plsc.cumsum(vp)
        starts[pl.ds(i * _L, _L)] = carry + ic - vp
        return carry + jnp.sum(vp)

    lax.fori_loop(0, CNT // _L, cum_body, jnp.int32(0))

    for j in range(CNT // _L):
        offs[pl.ds(j * _L, _L)] = starts[pl.ds(j * _L, _L)]

    def place_body(k, carry):
        v = idx_v[pl.ds(k * _L, _L)]
        lp = lax.shift_right_logical(v, 7) - lo
        m = (lp >= 0) & (lp < PT)
        o = plsc.load_gather(offs, [lp], mask=m)
        dup, lastm = plsc.scan_count(lp, mask=m)
        packed = lax.shift_left(v & 127, 14) | (k * _L + iota)
        plsc.store_scatter(bucket, [o + dup - 1], packed, mask=m)
        plsc.addupdate_scatter(offs, [lp], dup, mask=m & lastm)
        return carry

    lax.fori_loop(0, NCHUNK, place_body, jnp.int32(0))

    plsc.subcore_barrier()

    def issue(pg_global, slot):
        off = pl.multiple_of(pg_global * 128, 128)

        @pl.when(slot == 0)
        def _():
            @pl.when(c == 0)
            def _():
                pltpu.async_copy(kt_hbm.at[:, pl.ds(off, 128)],
                                 sb3.at[0], sem_a)

            @pl.when(c != 0)
            def _():
                pltpu.async_copy(vt_hbm.at[:, pl.ds(off, 128)],
                                 sb3.at[0], sem_a)

        @pl.when(slot != 0)
        def _():
            @pl.when(c == 0)
            def _():
                pltpu.async_copy(kt_hbm.at[:, pl.ds(off, 128)],
                                 sb3.at[1], sem_b)

            @pl.when(c != 0)
            def _():
                pltpu.async_copy(vt_hbm.at[:, pl.ds(off, 128)],
                                 sb3.at[1], sem_b)

    def wait_slot(slot):
        @pl.when(slot == 0)
        def _():
            pltpu.make_async_copy(kt_hbm.at[:, pl.ds(0, 128)], sb3.at[0],
                                  sem_a).wait()

        @pl.when(slot != 0)
        def _():
            pltpu.make_async_copy(kt_hbm.at[:, pl.ds(0, 128)], sb3.at[1],
                                  sem_b).wait()

    issue(jnp.minimum(lo, NPAGES - 1), jnp.int32(0))

    def page_body(p, carry):
        slot = lax.rem(p, 2)
        wait_slot(slot)
        nxt = jnp.minimum(lo + p + 1, NPAGES - 1)
        @pl.when(p + 1 < PT)
        def _():
            issue(nxt, 1 - slot)

        cnt = cnts[pl.ds(p, _L)][0]
        start = starts[pl.ds(p, _L)][0]

        @pl.when(cnt > 0)
        def _():
            slot_b = jnp.full((_L,), slot, jnp.int32)

            def chunk_body(q, carry2):
                base = pl.multiple_of(start + q * _L, _L)
                w = bucket[pl.ds(base, _L)]
                m = iota < (cnt - q * _L)
                col = lax.shift_right_logical(w, 14) & 127
                b = w & (BATCH - 1)
                b2 = jnp.where(m, lax.shift_right_logical(b, 1),
                               jnp.int32(ACCR))
                parbase = lax.bitwise_and(b, 1) * HIDDEN
                oppbase = HIDDEN - parbase
                zvec = jnp.zeros((_L,), jnp.float32)
                for h in range(HIDDEN):
                    vals = plsc.load_gather(
                        sb3, [slot_b, jnp.full((_L,), h, jnp.int32), col],
                        mask=m)
                    plsc.store_scatter(rows, [iota, parbase + h], vals,
                                       mask=m)
                    plsc.store_scatter(rows, [iota, oppbase + h], zvec,
                                       mask=m)
                pltpu.sync_copy(rows, acc.at[b2], add=True)
                return carry2

            lax.fori_loop(0, (cnt + _L - 1) // _L, chunk_body, jnp.int32(0))

        return carry

    lax.fori_loop(0, PT, page_body, jnp.int32(0))

    plsc.subcore_barrier()

    src = pl.multiple_of(s * rows_per_tile, rows_per_tile)
    pltpu.async_copy(acc.at[pl.ds(src, rows_per_tile), :],
                     part_hbm.at[c, pl.ds(src, rows_per_tile), :],
                     sem_o)
    pltpu.make_async_copy(acc.at[pl.ds(0, rows_per_tile), :],
                          part_hbm.at[0, pl.ds(0, rows_per_tile), :],
                          sem_o).wait()


def _add_body(a_ref, b_ref, o_ref):
    o_ref[...] = a_ref[...] + b_ref[...]


@jax.jit
def kernel(x, key_table, val_table):
    x_key = x[:, 0]
    x_val = x[:, 1]
    kt = key_table.T
    vt = val_table.T
    run = pl.kernel(
        _sc_body,
        mesh=plsc.VectorSubcoreMesh(core_axis_name="c", subcore_axis_name="s"),
        compiler_params=pltpu.CompilerParams(needs_layout_passes=False),
        out_type=jax.ShapeDtypeStruct((2, ACCR, 128), jnp.float32),
        scratch_types=[
            pltpu.VMEM((BATCH,), jnp.int32),
            pltpu.VMEM((BUCKET,), jnp.int32),
            pltpu.VMEM((CNT,), jnp.int32),
            pltpu.VMEM((CNT,), jnp.int32),
            pltpu.VMEM((CNT,), jnp.int32),
            pltpu.VMEM((_L, 128), jnp.float32),
            pltpu.VMEM((2, HIDDEN, 128), jnp.float32),
            pltpu.VMEM((_L, 128), jnp.float32),
            pltpu.VMEM_SHARED((ACCR_PAD, 128), jnp.float32),
            pltpu.SemaphoreType.DMA,
            pltpu.SemaphoreType.DMA,
            pltpu.SemaphoreType.DMA,
            pltpu.SemaphoreType.DMA,
        ],
    )
    parts = run(x_key, x_val, kt, vt)
    summed = pl.pallas_call(
        _add_body,
        grid=(16,),
        in_specs=[pl.BlockSpec((ACCR // 16, 128), lambda i: (i, 0))] * 2,
        out_specs=pl.BlockSpec((ACCR // 16, 128), lambda i: (i, 0)),
        out_shape=jax.ShapeDtypeStruct((ACCR, 128), jnp.float32),
    )(parts[0], parts[1])
    return summed.reshape(BATCH, HIDDEN)


# speedup vs baseline: 3.6271x; 1.8896x over previous
import jax
import jax.numpy as jnp
from jax import lax
from jax.experimental import pallas as pl
from jax.experimental.pallas import tpu as pltpu
from jax.experimental.pallas import tpu_sc as plsc

BATCH = 16384
HIDDEN = 64
DIM = 1000001

_info = plsc.get_sparse_core_info()
_NC, _NS, _L = _info.num_cores, _info.num_subcores, _info.num_lanes

NPAGES = (DIM + 127) // 128
PT = (NPAGES + _NS - 1) // _NS
SEG = 4096
NSEG = BATCH // SEG
BUCKET = BATCH + _L
ACCR = BATCH // 2
ACCR_PAD = ACCR + _L
CNT = 512
NSB = 4


def _sc_body(kidx_hbm, vidx_hbm, kt_hbm, vt_hbm, part_hbm,
             idx_v, bucket, cnts, starts, offs, rows, sb4, zbuf, b_stage,
             acc, sem_i, sem_s0, sem_s1, sem_s2, sem_s3, sem_o):
    c = lax.axis_index("c")
    s = lax.axis_index("s")
    iota = lax.iota(jnp.int32, _L)
    lo = s * PT
    sb_sems = (sem_s0, sem_s1, sem_s2, sem_s3)

    for h in range(_L):
        for j in range(128 // _L):
            zbuf[h, pl.ds(j * _L, _L)] = jnp.zeros((_L,), jnp.float32)
    rows_per_tile = ACCR // _NS
    for k in range(rows_per_tile // _L):
        dst = pl.multiple_of(s * rows_per_tile + k * _L, _L)
        pltpu.sync_copy(zbuf, acc.at[pl.ds(dst, _L), :])

    @pl.when(s == 0)
    def _():
        pltpu.sync_copy(zbuf, acc.at[pl.ds(ACCR, _L), :])

    for j in range(CNT // _L):
        cnts[pl.ds(j * _L, _L)] = jnp.zeros((_L,), jnp.int32)

    def stage(g):
        @pl.when(c == 0)
        def _():
            pltpu.async_copy(kidx_hbm.at[pl.ds(g * SEG, SEG)], idx_v, sem_i)

        @pl.when(c != 0)
        def _():
            pltpu.async_copy(vidx_hbm.at[pl.ds(g * SEG, SEG)], idx_v, sem_i)

        pltpu.make_async_copy(kidx_hbm.at[pl.ds(0, SEG)], idx_v, sem_i).wait()

    for g in range(NSEG):
        stage(g)

        def hist_body(k, carry):
            v = idx_v[pl.ds(k * _L, _L)]
            lp = lax.shift_right_logical(v, 7) - lo
            m = (lp >= 0) & (lp < PT)
            dup, lastm = plsc.scan_count(lp, mask=m)
            plsc.addupdate_scatter(cnts, [lp], dup, mask=m & lastm)
            return carry

        lax.fori_loop(0, SEG // _L, hist_body, jnp.int32(0))

    def cum_body(i, carry):
        v = cnts[pl.ds(i * _L, _L)]
        ic = plsc.cumsum(v)
        starts[pl.ds(i * _L, _L)] = carry + ic - v
        return carry + jnp.sum(v)

    lax.fori_loop(0, CNT // _L, cum_body, jnp.int32(0))

    for j in range(CNT // _L):
        offs[pl.ds(j * _L, _L)] = starts[pl.ds(j * _L, _L)]

    for g in range(NSEG):
        stage(g)

        def place_body(k, carry):
            v = idx_v[pl.ds(k * _L, _L)]
            lp = lax.shift_right_logical(v, 7) - lo
            m = (lp >= 0) & (lp < PT)
            o = plsc.load_gather(offs, [lp], mask=m)
            dup, lastm = plsc.scan_count(lp, mask=m)
            packed = lax.shift_left(v & 127, 14) | (g * SEG + k * _L + iota)
            plsc.store_scatter(bucket, [o + dup - 1], packed, mask=m)
            plsc.addupdate_scatter(offs, [lp], dup, mask=m & lastm)
            return carry

        lax.fori_loop(0, SEG // _L, place_body, jnp.int32(0))

    plsc.subcore_barrier()

    def issue(pg_global, slot):
        off = pl.multiple_of(pg_global * 128, 128)
        for q in range(NSB):
            @pl.when(slot == q)
            def _():
                @pl.when(c == 0)
                def _():
                    pltpu.async_copy(kt_hbm.at[:, pl.ds(off, 128)],
                                     sb4.at[q], sb_sems[q])

                @pl.when(c != 0)
                def _():
                    pltpu.async_copy(vt_hbm.at[:, pl.ds(off, 128)],
                                     sb4.at[q], sb_sems[q])

    def wait_slot(slot):
        for q in range(NSB):
            @pl.when(slot == q)
            def _():
                pltpu.make_async_copy(kt_hbm.at[:, pl.ds(0, 128)],
                                      sb4.at[q], sb_sems[q]).wait()

    for q in range(NSB):
        issue(jnp.minimum(lo + q, NPAGES - 1), jnp.int32(q))

    zvec = jnp.zeros((_L,), jnp.float32)

    def page_body(p, fill):
        slot = lax.rem(p, NSB)
        wait_slot(slot)
        cnt = cnts[pl.ds(p, _L)][0]
        start = starts[pl.ds(p, _L)][0]
        slot_b = jnp.full((_L,), slot, jnp.int32)

        def elem_body(e, f):
            w = bucket[pl.ds(start + e, _L)][0]
            col = lax.shift_right_logical(w, 14) & 127
            b = w & (BATCH - 1)
            parbase = lax.bitwise_and(b, 1) * HIDDEN
            oppbase = HIDDEN - parbase
            colv = jnp.full((_L,), col, jnp.int32)
            for j in range(HIDDEN // _L):
                vals = plsc.load_gather(sb4, [slot_b, iota + j * _L, colv])
                rows[f, pl.ds(parbase + j * _L, _L)] = vals
                rows[f, pl.ds(oppbase + j * _L, _L)] = zvec
            plsc.store_scatter(
                b_stage, [jnp.full((_L,), f, jnp.int32)],
                jnp.full((_L,), lax.shift_right_logical(b, 1), jnp.int32),
                mask=iota == 0)
            nf = f + 1

            @pl.when(nf == _L)
            def _():
                pltpu.sync_copy(rows, acc.at[b_stage], add=True)

            return jnp.where(nf == _L, 0, nf)

        fill = lax.fori_loop(0, cnt, elem_body, fill)
        nxt = jnp.minimum(lo + p + NSB, NPAGES - 1)

        @pl.when(p + NSB < PT)
        def _():
            issue(nxt, slot)

        return fill

    fill = lax.fori_loop(0, PT, page_body, jnp.int32(0))

    @pl.when(fill > 0)
    def _():
        plsc.store_scatter(b_stage, [iota],
                           jnp.full((_L,), ACCR, jnp.int32),
                           mask=iota >= fill)
        pltpu.sync_copy(rows, acc.at[b_stage], add=True)

    plsc.subcore_barrier()

    src = pl.multiple_of(s * rows_per_tile, rows_per_tile)
    pltpu.async_copy(acc.at[pl.ds(src, rows_per_tile), :],
                     part_hbm.at[c, pl.ds(src, rows_per_tile), :],
                     sem_o)
    pltpu.make_async_copy(acc.at[pl.ds(0, rows_per_tile), :],
                          part_hbm.at[0, pl.ds(0, rows_per_tile), :],
                          sem_o).wait()


def _add_body(a_ref, b_ref, o_ref):
    o_ref[...] = a_ref[...] + b_ref[...]


@jax.jit
def kernel(x, key_table, val_table):
    x_key = x[:, 0]
    x_val = x[:, 1]
    kt = key_table.T
    vt = val_table.T
    run = pl.kernel(
        _sc_body,
        mesh=plsc.VectorSubcoreMesh(core_axis_name="c", subcore_axis_name="s"),
        compiler_params=pltpu.CompilerParams(needs_layout_passes=False),
        out_type=jax.ShapeDtypeStruct((2, ACCR, 128), jnp.float32),
        scratch_types=[
            pltpu.VMEM((SEG,), jnp.int32),
            pltpu.VMEM((BUCKET,), jnp.int32),
            pltpu.VMEM((CNT,), jnp.int32),
            pltpu.VMEM((CNT,), jnp.int32),
            pltpu.VMEM((CNT,), jnp.int32),
            pltpu.VMEM((_L, 128), jnp.float32),
            pltpu.VMEM((NSB, HIDDEN, 128), jnp.float32),
            pltpu.VMEM((_L, 128), jnp.float32),
            pltpu.VMEM((_L,), jnp.int32),
            pltpu.VMEM_SHARED((ACCR_PAD, 128), jnp.float32),
            pltpu.SemaphoreType.DMA,
            pltpu.SemaphoreType.DMA,
            pltpu.SemaphoreType.DMA,
            pltpu.SemaphoreType.DMA,
            pltpu.SemaphoreType.DMA,
            pltpu.SemaphoreType.DMA,
        ],
    )
    parts = run(x_key, x_val, kt, vt)
    summed = pl.pallas_call(
        _add_body,
        grid=(16,),
        in_specs=[pl.BlockSpec((ACCR // 16, 128), lambda i: (i, 0))] * 2,
        out_specs=pl.BlockSpec((ACCR // 16, 128), lambda i: (i, 0)),
        out_shape=jax.ShapeDtypeStruct((ACCR, 128), jnp.float32),
    )(parts[0], parts[1])
    return summed.reshape(BATCH, HIDDEN)


# speedup vs baseline: 4.0837x; 1.1259x over previous
import jax
import jax.numpy as jnp
from jax import lax
from jax.experimental import pallas as pl
from jax.experimental.pallas import tpu as pltpu
from jax.experimental.pallas import tpu_sc as plsc

BATCH = 16384
HIDDEN = 64
DIM = 1000001

_info = plsc.get_sparse_core_info()
_NC, _NS, _L = _info.num_cores, _info.num_subcores, _info.num_lanes

NPAGES = (DIM + 127) // 128
PT = (NPAGES + _NS - 1) // _NS
SEG = 4096
NSEG = BATCH // SEG
BUCKET = BATCH + _L
ACCR = BATCH // 2
ACCR_PAD = ACCR + _L
CNT = 512
NSB = 4


def _sc_body(kidx_hbm, vidx_hbm, kt_hbm, vt_hbm, part_hbm,
             idx_v, bucket, cnts, starts, offs, nzlist, rows, sb4, zbuf,
             b_stage, acc, sem_i, sem_s0, sem_s1, sem_s2, sem_s3, sem_o):
    c = lax.axis_index("c")
    s = lax.axis_index("s")
    iota = lax.iota(jnp.int32, _L)
    lo = s * PT
    sb_sems = (sem_s0, sem_s1, sem_s2, sem_s3)

    for h in range(_L):
        for j in range(128 // _L):
            zbuf[h, pl.ds(j * _L, _L)] = jnp.zeros((_L,), jnp.float32)
    rows_per_tile = ACCR // _NS
    for k in range(rows_per_tile // _L):
        dst = pl.multiple_of(s * rows_per_tile + k * _L, _L)
        pltpu.sync_copy(zbuf, acc.at[pl.ds(dst, _L), :])

    @pl.when(s == 0)
    def _():
        pltpu.sync_copy(zbuf, acc.at[pl.ds(ACCR, _L), :])

    for j in range(CNT // _L):
        cnts[pl.ds(j * _L, _L)] = jnp.zeros((_L,), jnp.int32)

    def stage(g):
        @pl.when(c == 0)
        def _():
            pltpu.async_copy(kidx_hbm.at[pl.ds(g * SEG, SEG)], idx_v, sem_i)

        @pl.when(c != 0)
        def _():
            pltpu.async_copy(vidx_hbm.at[pl.ds(g * SEG, SEG)], idx_v, sem_i)

        pltpu.make_async_copy(kidx_hbm.at[pl.ds(0, SEG)], idx_v, sem_i).wait()

    for g in range(NSEG):
        stage(g)

        def hist_body(k, carry):
            v = idx_v[pl.ds(k * _L, _L)]
            lp = lax.shift_right_logical(v, 7) - lo
            m = (lp >= 0) & (lp < PT)
            plsc.addupdate_scatter(cnts, [lp], jnp.ones((_L,), jnp.int32),
                                   mask=m)
            return carry

        lax.fori_loop(0, SEG // _L, hist_body, jnp.int32(0))

    def cum_body(i, carry):
        v = cnts[pl.ds(i * _L, _L)]
        ic = plsc.cumsum(v)
        starts[pl.ds(i * _L, _L)] = carry + ic - v
        return carry + jnp.sum(v)

    lax.fori_loop(0, CNT // _L, cum_body, jnp.int32(0))

    for j in range(CNT // _L):
        offs[pl.ds(j * _L, _L)] = starts[pl.ds(j * _L, _L)]

    def nz_body(i, np_):
        v = cnts[pl.ds(i * _L, _L)]
        m = v > 0
        pf = plsc.cumsum(m.astype(jnp.int32))
        plsc.store_scatter(nzlist, [np_ + pf - 1], i * _L + iota, mask=m)
        return np_ + jnp.sum(m.astype(jnp.int32))

    npg = lax.fori_loop(0, CNT // _L, nz_body, jnp.int32(0))

    for g in range(NSEG):
        stage(g)

        def place_body(k, carry):
            v = idx_v[pl.ds(k * _L, _L)]
            lp = lax.shift_right_logical(v, 7) - lo
            m = (lp >= 0) & (lp < PT)
            o = plsc.load_gather(offs, [lp], mask=m)
            dup, lastm = plsc.scan_count(lp, mask=m)
            packed = lax.shift_left(v & 127, 14) | (g * SEG + k * _L + iota)
            plsc.store_scatter(bucket, [o + dup - 1], packed, mask=m)
            plsc.addupdate_scatter(offs, [lp], dup, mask=m & lastm)
            return carry

        lax.fori_loop(0, SEG // _L, place_body, jnp.int32(0))

    plsc.subcore_barrier()

    def nz_at(i):
        ii = jnp.maximum(jnp.minimum(i, npg - 1), 0)
        pgl = nzlist[pl.ds(ii, _L)][0]
        return jnp.maximum(jnp.minimum(pgl, PT - 1), 0)

    def issue(pg_global, slot):
        off = pl.multiple_of(pg_global * 128, 128)
        for q in range(NSB):
            @pl.when(slot == q)
            def _():
                @pl.when(c == 0)
                def _():
                    pltpu.async_copy(kt_hbm.at[:, pl.ds(off, 128)],
                                     sb4.at[q], sb_sems[q])

                @pl.when(c != 0)
                def _():
                    pltpu.async_copy(vt_hbm.at[:, pl.ds(off, 128)],
                                     sb4.at[q], sb_sems[q])

    def wait_slot(slot):
        for q in range(NSB):
            @pl.when(slot == q)
            def _():
                pltpu.make_async_copy(kt_hbm.at[:, pl.ds(0, 128)],
                                      sb4.at[q], sb_sems[q]).wait()

    for q in range(NSB):
        issue(jnp.minimum(lo + nz_at(jnp.int32(q)), NPAGES - 1),
              jnp.int32(q))

    zvec = jnp.zeros((_L,), jnp.float32)

    def page_body(i, fill):
        slot = lax.rem(i, NSB)
        wait_slot(slot)
        p = nz_at(i)
        cnt = cnts[pl.ds(p, _L)][0]
        start = starts[pl.ds(p, _L)][0]
        slot_b = jnp.full((_L,), slot, jnp.int32)

        def elem_body(e, f):
            w = bucket[pl.ds(start + e, _L)][0]
            col = lax.shift_right_logical(w, 14) & 127
            b = w & (BATCH - 1)
            parbase = lax.bitwise_and(b, 1) * HIDDEN
            oppbase = HIDDEN - parbase
            colv = jnp.full((_L,), col, jnp.int32)
            for j in range(HIDDEN // _L):
                vals = plsc.load_gather(sb4, [slot_b, iota + j * _L, colv])
                rows[f, pl.ds(parbase + j * _L, _L)] = vals
                rows[f, pl.ds(oppbase + j * _L, _L)] = zvec
            plsc.store_scatter(
                b_stage, [jnp.full((_L,), f, jnp.int32)],
                jnp.full((_L,), lax.shift_right_logical(b, 1), jnp.int32),
                mask=iota == 0)
            nf = f + 1

            @pl.when(nf == _L)
            def _():
                pltpu.sync_copy(rows, acc.at[b_stage], add=True)

            return jnp.where(nf == _L, 0, nf)

        fill = lax.fori_loop(0, cnt, elem_body, fill)
        nxt = jnp.minimum(lo + nz_at(i + NSB), NPAGES - 1)

        @pl.when(i + NSB < npg)
        def _():
            issue(nxt, slot)

        return fill

    fill = lax.fori_loop(0, npg, page_body, jnp.int32(0))

    @pl.when(fill > 0)
    def _():
        plsc.store_scatter(b_stage, [iota],
                           jnp.full((_L,), ACCR, jnp.int32),
                           mask=iota >= fill)
        pltpu.sync_copy(rows, acc.at[b_stage], add=True)

    plsc.subcore_barrier()

    src = pl.multiple_of(s * rows_per_tile, rows_per_tile)
    pltpu.async_copy(acc.at[pl.ds(src, rows_per_tile), :],
                     part_hbm.at[c, pl.ds(src, rows_per_tile), :],
                     sem_o)
    pltpu.make_async_copy(acc.at[pl.ds(0, rows_per_tile), :],
                          part_hbm.at[0, pl.ds(0, rows_per_tile), :],
                          sem_o).wait()


def _add_body(a_ref, b_ref, o_ref):
    o_ref[...] = a_ref[...] + b_ref[...]


@jax.jit
def kernel(x, key_table, val_table):
    x_key = x[:, 0]
    x_val = x[:, 1]
    kt = key_table.T
    vt = val_table.T
    run = pl.kernel(
        _sc_body,
        mesh=plsc.VectorSubcoreMesh(core_axis_name="c", subcore_axis_name="s"),
        compiler_params=pltpu.CompilerParams(needs_layout_passes=False),
        out_type=jax.ShapeDtypeStruct((2, ACCR, 128), jnp.float32),
        scratch_types=[
            pltpu.VMEM((SEG,), jnp.int32),
            pltpu.VMEM((BUCKET,), jnp.int32),
            pltpu.VMEM((CNT,), jnp.int32),
            pltpu.VMEM((CNT,), jnp.int32),
            pltpu.VMEM((CNT,), jnp.int32),
            pltpu.VMEM((CNT,), jnp.int32),
            pltpu.VMEM((_L, 128), jnp.float32),
            pltpu.VMEM((NSB, HIDDEN, 128), jnp.float32),
            pltpu.VMEM((_L, 128), jnp.float32),
            pltpu.VMEM((_L,), jnp.int32),
            pltpu.VMEM_SHARED((ACCR_PAD, 128), jnp.float32),
            pltpu.SemaphoreType.DMA,
            pltpu.SemaphoreType.DMA,
            pltpu.SemaphoreType.DMA,
            pltpu.SemaphoreType.DMA,
            pltpu.SemaphoreType.DMA,
            pltpu.SemaphoreType.DMA,
        ],
    )
    parts = run(x_key, x_val, kt, vt)
    summed = pl.pallas_call(
        _add_body,
        grid=(16,),
        in_specs=[pl.BlockSpec((ACCR // 16, 128), lambda i: (i, 0))] * 2,
        out_specs=pl.BlockSpec((ACCR // 16, 128), lambda i: (i, 0)),
        out_shape=jax.ShapeDtypeStruct((ACCR, 128), jnp.float32),
    )(parts[0], parts[1])
    return summed.reshape(BATCH, HIDDEN)


# speedup vs baseline: 4.2811x; 1.0483x over previous
import jax
import jax.numpy as jnp
from jax import lax
from jax.experimental import pallas as pl
from jax.experimental.pallas import tpu as pltpu
from jax.experimental.pallas import tpu_sc as plsc

BATCH = 16384
HIDDEN = 64
DIM = 1000001

_info = plsc.get_sparse_core_info()
_NC, _NS, _L = _info.num_cores, _info.num_subcores, _info.num_lanes

NPAGES = (DIM + 127) // 128
PT = (NPAGES + _NS - 1) // _NS
SEG = 2048
NSEG = BATCH // SEG
BUCKET = BATCH + _L
ACCR = BATCH // 2
ACCR_PAD = ACCR + _L
CNT = 512
NSB = 5


def _sc_body(kidx_hbm, vidx_hbm, kt_hbm, vt_hbm, part_hbm,
             idx_v, bucket, cnts, starts, offs, nzlist, rows, sb4,
             b_stage, acc, sem_i, sem_s0, sem_s1, sem_s2, sem_s3, sem_s4,
             sem_o):
    c = lax.axis_index("c")
    s = lax.axis_index("s")
    iota = lax.iota(jnp.int32, _L)
    lo = s * PT
    sb_sems = (sem_s0, sem_s1, sem_s2, sem_s3, sem_s4)

    for h in range(_L):
        for j in range(128 // _L):
            rows[h, pl.ds(j * _L, _L)] = jnp.zeros((_L,), jnp.float32)
    rows_per_tile = ACCR // _NS
    for k in range(rows_per_tile // _L):
        dst = pl.multiple_of(s * rows_per_tile + k * _L, _L)
        pltpu.sync_copy(rows, acc.at[pl.ds(dst, _L), :])

    @pl.when(s == 0)
    def _():
        pltpu.sync_copy(rows, acc.at[pl.ds(ACCR, _L), :])

    for j in range(CNT // _L):
        cnts[pl.ds(j * _L, _L)] = jnp.zeros((_L,), jnp.int32)

    def stage(g):
        @pl.when(c == 0)
        def _():
            pltpu.async_copy(kidx_hbm.at[pl.ds(g * SEG, SEG)], idx_v, sem_i)

        @pl.when(c != 0)
        def _():
            pltpu.async_copy(vidx_hbm.at[pl.ds(g * SEG, SEG)], idx_v, sem_i)

        pltpu.make_async_copy(kidx_hbm.at[pl.ds(0, SEG)], idx_v, sem_i).wait()

    for g in range(NSEG):
        stage(g)

        def hist_body(k, carry):
            v = idx_v[pl.ds(k * _L, _L)]
            lp = lax.shift_right_logical(v, 7) - lo
            m = (lp >= 0) & (lp < PT)
            plsc.addupdate_scatter(cnts, [lp], jnp.ones((_L,), jnp.int32),
                                   mask=m)
            return carry

        lax.fori_loop(0, SEG // _L, hist_body, jnp.int32(0))

    def cum_body(i, carry):
        v = cnts[pl.ds(i * _L, _L)]
        ic = plsc.cumsum(v)
        starts[pl.ds(i * _L, _L)] = carry + ic - v
        return carry + jnp.sum(v)

    lax.fori_loop(0, CNT // _L, cum_body, jnp.int32(0))

    for j in range(CNT // _L):
        offs[pl.ds(j * _L, _L)] = starts[pl.ds(j * _L, _L)]

    def nz_body(i, np_):
        v = cnts[pl.ds(i * _L, _L)]
        m = v > 0
        pf = plsc.cumsum(m.astype(jnp.int32))
        plsc.store_scatter(nzlist, [np_ + pf - 1], i * _L + iota, mask=m)
        return np_ + jnp.sum(m.astype(jnp.int32))

    npg = lax.fori_loop(0, CNT // _L, nz_body, jnp.int32(0))

    for g in range(NSEG):
        stage(g)

        def place_body(k, carry):
            v = idx_v[pl.ds(k * _L, _L)]
            lp = lax.shift_right_logical(v, 7) - lo
            m = (lp >= 0) & (lp < PT)
            o = plsc.load_gather(offs, [lp], mask=m)
            dup, lastm = plsc.scan_count(lp, mask=m)
            packed = lax.shift_left(v & 127, 14) | (g * SEG + k * _L + iota)
            plsc.store_scatter(bucket, [o + dup - 1], packed, mask=m)
            plsc.addupdate_scatter(offs, [lp], dup, mask=m & lastm)
            return carry

        lax.fori_loop(0, SEG // _L, place_body, jnp.int32(0))

    plsc.subcore_barrier()

    def nz_at(i):
        ii = jnp.maximum(jnp.minimum(i, npg - 1), 0)
        pgl = nzlist[pl.ds(ii, _L)][0]
        return jnp.maximum(jnp.minimum(pgl, PT - 1), 0)

    def issue(pg_global, slot):
        off = pl.multiple_of(pg_global * 128, 128)
        for q in range(NSB):
            @pl.when(slot == q)
            def _():
                @pl.when(c == 0)
                def _():
                    pltpu.async_copy(kt_hbm.at[:, pl.ds(off, 128)],
                                     sb4.at[q], sb_sems[q])

                @pl.when(c != 0)
                def _():
                    pltpu.async_copy(vt_hbm.at[:, pl.ds(off, 128)],
                                     sb4.at[q], sb_sems[q])

    def wait_slot(slot):
        for q in range(NSB):
            @pl.when(slot == q)
            def _():
                pltpu.make_async_copy(kt_hbm.at[:, pl.ds(0, 128)],
                                      sb4.at[q], sb_sems[q]).wait()

    for q in range(NSB):
        issue(jnp.minimum(lo + nz_at(jnp.int32(q)), NPAGES - 1),
              jnp.int32(q))

    zvec = jnp.zeros((_L,), jnp.float32)

    def page_body(i, fill):
        slot = lax.rem(i, NSB)
        wait_slot(slot)
        p = nz_at(i)
        cnt = cnts[pl.ds(p, _L)][0]
        start = starts[pl.ds(p, _L)][0]
        slot_b = jnp.full((_L,), slot, jnp.int32)

        def elem_body(e, f):
            w = bucket[pl.ds(start + e, _L)][0]
            col = lax.shift_right_logical(w, 14) & 127
            b = w & (BATCH - 1)
            parbase = lax.bitwise_and(b, 1) * HIDDEN
            oppbase = HIDDEN - parbase
            colv = jnp.full((_L,), col, jnp.int32)
            for j in range(HIDDEN // _L):
                vals = plsc.load_gather(sb4, [slot_b, iota + j * _L, colv])
                rows[f, pl.ds(parbase + j * _L, _L)] = vals
                rows[f, pl.ds(oppbase + j * _L, _L)] = zvec
            plsc.store_scatter(
                b_stage, [jnp.full((_L,), f, jnp.int32)],
                jnp.full((_L,), lax.shift_right_logical(b, 1), jnp.int32),
                mask=iota == 0)
            nf = f + 1

            @pl.when(nf == _L)
            def _():
                pltpu.sync_copy(rows, acc.at[b_stage], add=True)

            return jnp.where(nf == _L, 0, nf)

        fill = lax.fori_loop(0, cnt, elem_body, fill)
        nxt = jnp.minimum(lo + nz_at(i + NSB), NPAGES - 1)

        @pl.when(i + NSB < npg)
        def _():
            issue(nxt, slot)

        return fill

    fill = lax.fori_loop(0, npg, page_body, jnp.int32(0))

    @pl.when(fill > 0)
    def _():
        plsc.store_scatter(b_stage, [iota],
                           jnp.full((_L,), ACCR, jnp.int32),
                           mask=iota >= fill)
        pltpu.sync_copy(rows, acc.at[b_stage], add=True)

    plsc.subcore_barrier()

    src = pl.multiple_of(s * rows_per_tile, rows_per_tile)
    pltpu.async_copy(acc.at[pl.ds(src, rows_per_tile), :],
                     part_hbm.at[c, pl.ds(src, rows_per_tile), :],
                     sem_o)
    pltpu.make_async_copy(acc.at[pl.ds(0, rows_per_tile), :],
                          part_hbm.at[0, pl.ds(0, rows_per_tile), :],
                          sem_o).wait()


def _add_body(a_ref, b_ref, o_ref):
    o_ref[...] = a_ref[...] + b_ref[...]


@jax.jit
def kernel(x, key_table, val_table):
    x_key = x[:, 0]
    x_val = x[:, 1]
    kt = key_table.T
    vt = val_table.T
    run = pl.kernel(
        _sc_body,
        mesh=plsc.VectorSubcoreMesh(core_axis_name="c", subcore_axis_name="s"),
        compiler_params=pltpu.CompilerParams(needs_layout_passes=False),
        out_type=jax.ShapeDtypeStruct((2, ACCR, 128), jnp.float32),
        scratch_types=[
            pltpu.VMEM((SEG,), jnp.int32),
            pltpu.VMEM((BUCKET,), jnp.int32),
            pltpu.VMEM((CNT,), jnp.int32),
            pltpu.VMEM((CNT,), jnp.int32),
            pltpu.VMEM((CNT,), jnp.int32),
            pltpu.VMEM((CNT,), jnp.int32),
            pltpu.VMEM((_L, 128), jnp.float32),
            pltpu.VMEM((NSB, HIDDEN, 128), jnp.float32),
            pltpu.VMEM((_L,), jnp.int32),
            pltpu.VMEM_SHARED((ACCR_PAD, 128), jnp.float32),
            pltpu.SemaphoreType.DMA,
            pltpu.SemaphoreType.DMA,
            pltpu.SemaphoreType.DMA,
            pltpu.SemaphoreType.DMA,
            pltpu.SemaphoreType.DMA,
            pltpu.SemaphoreType.DMA,
            pltpu.SemaphoreType.DMA,
        ],
    )
    parts = run(x_key, x_val, kt, vt)
    summed = pl.pallas_call(
        _add_body,
        grid=(16,),
        in_specs=[pl.BlockSpec((ACCR // 16, 128), lambda i: (i, 0))] * 2,
        out_specs=pl.BlockSpec((ACCR // 16, 128), lambda i: (i, 0)),
        out_shape=jax.ShapeDtypeStruct((ACCR, 128), jnp.float32),
    )(parts[0], parts[1])
    return summed.reshape(BATCH, HIDDEN)
